# baseline (speedup 1.0000x reference)
import jax
import jax.numpy as jnp
from jax import lax
from jax.experimental import pallas as pl
from jax.experimental.pallas import tpu as pltpu
from jax.experimental.pallas import tpu_sc as plsc

B = 32
C_IN = 3
H = 512
W = 512
CH = 448
CW = 448
NUM_CROPS = 2

NW = 32
TRIPLES = B * NUM_CROPS * C_IN
PER_W = TRIPLES // NW
KR = 56
NCHUNK = CH // KR
NU = PER_W * NCHUNK
LANES = 16
NJ = CW // LANES
WIN = CW + 8


def _body(inds_hbm, in_hbm, out_hbm, inds_v,
          ib0, ib1, ob0, ob1, isem0, isem1, osem0, osem1):
    wid = lax.axis_index("s") * 2 + lax.axis_index("c")
    pltpu.sync_copy(inds_hbm, inds_v.at[pl.ds(0, B * NUM_CROPS * 2)])
    lanes = lax.iota(jnp.int32, LANES)
    ibufs, obufs = (ib0, ib1), (ob0, ob1)
    isems, osems = (isem0, isem1), (osem0, osem1)

    def params(u):
        j = u // NCHUNK
        k = u % NCHUNK
        t = wid * PER_W + j
        b = t // (NUM_CROPS * C_IN)
        r = t % (NUM_CROPS * C_IN)
        n = r // C_IN
        c = r % C_IN
        hw = inds_v[pl.ds((b * NUM_CROPS + n) * 2, LANES)]
        h0 = hw[0]
        w0 = hw[1]
        w8 = pl.multiple_of((w0 // 8) * 8, 8)
        return b, n, c, k, h0, w0 - w8, w8

    def in_copy(u, s):
        b, _, c, k, h0, _, w8 = params(u)
        return pltpu.make_async_copy(
            in_hbm.at[b, c, pl.ds(h0 + k * KR, KR), pl.ds(w8, WIN)],
            ibufs[s], isems[s])

    def out_copy(u, s):
        b, n, c, k, _, _, _ = params(u)
        return pltpu.make_async_copy(
            obufs[s],
            out_hbm.at[b * NUM_CROPS + n, c, pl.ds(k * KR, KR), :],
            osems[s])

    def compute(u, s):
        _, _, _, _, _, d, _ = params(u)
        ib, ob = ibufs[s], obufs[s]
        col0 = d + lanes

        @plsc.parallel_loop(0, KR, unroll=2)
        def _(rr):
            row = jnp.full((LANES,), rr, jnp.int32)
            for jj in range(NJ):
                v = plsc.load_gather(ib, [row, col0 + jj * LANES])
                ob[rr, pl.ds(jj * LANES, LANES)] = v

    in_copy(0, 0).start()

    def pair_body(p, _):
        for s in range(2):
            u = 2 * p + s

            @pl.when(u + 1 < NU)
            def _():
                in_copy(u + 1, (s + 1) % 2).start()

            in_copy(u, s).wait()

            @pl.when(u >= 2)
            def _():
                out_copy(u - 2, s).wait()

            compute(u, s)
            out_copy(u, s).start()
        return 0

    lax.fori_loop(0, NU // 2, pair_body, 0)
    out_copy(NU - 2, 0).wait()
    out_copy(NU - 1, 1).wait()


def kernel(inputs, crop_inds):
    mesh = plsc.VectorSubcoreMesh(core_axis_name="c", subcore_axis_name="s",
                                  num_cores=2, num_subcores=16)
    f = pl.kernel(
        _body,
        out_type=jax.ShapeDtypeStruct((B * NUM_CROPS, C_IN, CH, CW),
                                      jnp.float32),
        mesh=mesh,
        compiler_params=pltpu.CompilerParams(use_tc_tiling_on_sc=False,
                                             needs_layout_passes=False),
        scratch_types=[
            pltpu.VMEM((B * NUM_CROPS * 2 + LANES,), jnp.int32),
            pltpu.VMEM((KR, WIN), jnp.float32),
            pltpu.VMEM((KR, WIN), jnp.float32),
            pltpu.VMEM((KR, CW), jnp.float32),
            pltpu.VMEM((KR, CW), jnp.float32),
            pltpu.SemaphoreType.DMA,
            pltpu.SemaphoreType.DMA,
            pltpu.SemaphoreType.DMA,
            pltpu.SemaphoreType.DMA,
        ],
    )
    return f(crop_inds.reshape(-1).astype(jnp.int32), inputs)


# speedup vs baseline: 2.0132x; 2.0132x over previous
import jax
import jax.numpy as jnp
from jax import lax
from jax.experimental import pallas as pl
from jax.experimental.pallas import tpu as pltpu
from jax.experimental.pallas import tpu_sc as plsc

B = 32
C_IN = 3
H = 512
W = 512
CH = 448
CW = 448
NUM_CROPS = 2

NW = 32
TRIPLES = B * NUM_CROPS * C_IN
PER_W = TRIPLES // NW
KR = 56
IR = KR + 8
NCHUNK = CH // KR
NU = PER_W * NCHUNK
LANES = 16
NJ = CW // LANES


def _body(inds_hbm, in_hbm, out_hbm, inds_v,
          ib0, ib1, ob0, ob1, isem0, isem1, osem0, osem1):
    wid = lax.axis_index("s") * 2 + lax.axis_index("c")
    pltpu.sync_copy(inds_hbm, inds_v.at[pl.ds(0, B * NUM_CROPS * 2)])
    lanes = lax.iota(jnp.int32, LANES)
    ibufs, obufs = (ib0, ib1), (ob0, ob1)
    isems, osems = (isem0, isem1), (osem0, osem1)

    def params(u):
        j = u // NCHUNK
        k = u % NCHUNK
        t = wid * PER_W + j
        b = t // (NUM_CROPS * C_IN)
        r = t % (NUM_CROPS * C_IN)
        n = r // C_IN
        c = r % C_IN
        hw = inds_v[pl.ds((b * NUM_CROPS + n) * 2, LANES)]
        h0 = hw[0]
        w0 = hw[1]
        h8 = pl.multiple_of((h0 // 8) * 8, 8)
        return b, n, c, k, h8, h0 - h8, w0

    def in_copy(u, s):
        b, _, c, k, h8, _, _ = params(u)
        return pltpu.make_async_copy(
            in_hbm.at[b, c, pl.ds(h8 + k * KR, IR), :],
            ibufs[s], isems[s])

    def out_copy(u, s):
        b, n, c, k, _, _, _ = params(u)
        return pltpu.make_async_copy(
            obufs[s],
            out_hbm.at[b * NUM_CROPS + n, c, pl.ds(k * KR, KR), :],
            osems[s])

    def compute(u, s):
        _, _, _, _, _, dh, w0 = params(u)
        ib, ob = ibufs[s], obufs[s]
        col0 = w0 + lanes

        @plsc.parallel_loop(0, KR, unroll=2)
        def _(rr):
            row = jnp.full((LANES,), rr + dh, jnp.int32)
            for jj in range(NJ):
                v = plsc.load_gather(ib, [row, col0 + jj * LANES])
                ob[rr, pl.ds(jj * LANES, LANES)] = v

    in_copy(0, 0).start()

    def pair_body(p, _):
        for s in range(2):
            u = 2 * p + s

            @pl.when(u + 1 < NU)
            def _():
                in_copy(u + 1, (s + 1) % 2).start()

            in_copy(u, s).wait()

            @pl.when(u >= 2)
            def _():
                out_copy(u - 2, s).wait()

            compute(u, s)
            out_copy(u, s).start()
        return 0

    lax.fori_loop(0, NU // 2, pair_body, 0)
    out_copy(NU - 2, 0).wait()
    out_copy(NU - 1, 1).wait()


def kernel(inputs, crop_inds):
    mesh = plsc.VectorSubcoreMesh(core_axis_name="c", subcore_axis_name="s",
                                  num_cores=2, num_subcores=16)
    f = pl.kernel(
        _body,
        out_type=jax.ShapeDtypeStruct((B * NUM_CROPS, C_IN, CH, CW),
                                      jnp.float32),
        mesh=mesh,
        compiler_params=pltpu.CompilerParams(use_tc_tiling_on_sc=True,
                                             needs_layout_passes=False),
        scratch_types=[
            pltpu.VMEM((B * NUM_CROPS * 2 + LANES,), jnp.int32),
            pltpu.VMEM((IR, W), jnp.float32),
            pltpu.VMEM((IR, W), jnp.float32),
            pltpu.VMEM((KR, CW), jnp.float32),
            pltpu.VMEM((KR, CW), jnp.float32),
            pltpu.SemaphoreType.DMA,
            pltpu.SemaphoreType.DMA,
            pltpu.SemaphoreType.DMA,
            pltpu.SemaphoreType.DMA,
        ],
    )
    return f(crop_inds.reshape(-1).astype(jnp.int32), inputs)


# speedup vs baseline: 2.2209x; 1.1031x over previous
import jax
import jax.numpy as jnp
from jax import lax
from jax.experimental import pallas as pl
from jax.experimental.pallas import tpu as pltpu
from jax.experimental.pallas import tpu_sc as plsc

B = 32
C_IN = 3
H = 512
W = 512
CH = 448
CW = 448
NUM_CROPS = 2

NW = 32
TRIPLES = B * NUM_CROPS * C_IN
PER_W = TRIPLES // NW
KR = 56
IR = KR + 8
NCHUNK = CH // KR
NU = PER_W * NCHUNK
LANES = 16
NJ = CW // LANES


def _body(inds_hbm, in_hbm, out_hbm, inds_v,
          ib0, ib1, ob0, ob1, isem0, isem1, osem0, osem1):
    wid = lax.axis_index("s") * 2 + lax.axis_index("c")
    pltpu.sync_copy(inds_hbm, inds_v.at[pl.ds(0, B * NUM_CROPS * 2)])
    lanes = lax.iota(jnp.int32, LANES)
    ibufs, obufs = (ib0, ib1), (ob0, ob1)
    isems, osems = (isem0, isem1), (osem0, osem1)

    def params(u):
        j = u // NCHUNK
        k = u % NCHUNK
        t = wid * PER_W + j
        b = t // (NUM_CROPS * C_IN)
        r = t % (NUM_CROPS * C_IN)
        n = r // C_IN
        c = r % C_IN
        hw = inds_v[pl.ds((b * NUM_CROPS + n) * 2, LANES)]
        h0 = hw[0]
        w0 = hw[1]
        h8 = pl.multiple_of((h0 // 8) * 8, 8)
        return b, n, c, k, h8, h0 - h8, w0

    def in_copy(u, s):
        b, _, c, k, h8, _, _ = params(u)
        return pltpu.make_async_copy(
            in_hbm.at[b, c, pl.ds(h8 + k * KR, IR), :],
            ibufs[s], isems[s])

    def out_copy(u, s):
        b, n, c, k, _, _, _ = params(u)
        return pltpu.make_async_copy(
            obufs[s],
            out_hbm.at[b * NUM_CROPS + n, c, pl.ds(k * KR, KR), :],
            osems[s])

    def compute(u, s):
        _, _, _, _, _, dh, w0 = params(u)
        ib, ob = ibufs[s], obufs[s]
        col0 = w0 + lanes

        @plsc.parallel_loop(0, KR, unroll=4)
        def _(rr):
            row = jnp.full((LANES,), rr + dh, jnp.int32)
            for jj in range(NJ):
                v = plsc.load_gather(ib, [row, col0 + jj * LANES])
                ob[rr, pl.ds(jj * LANES, LANES)] = v

    in_copy(0, 0).start()

    def pair_body(p, _):
        for s in range(2):
            u = 2 * p + s

            @pl.when(u + 1 < NU)
            def _():
                in_copy(u + 1, (s + 1) % 2).start()

            in_copy(u, s).wait()

            @pl.when(u >= 2)
            def _():
                out_copy(u - 2, s).wait()

            compute(u, s)
            out_copy(u, s).start()
        return 0

    lax.fori_loop(0, NU // 2, pair_body, 0)
    out_copy(NU - 2, 0).wait()
    out_copy(NU - 1, 1).wait()


def kernel(inputs, crop_inds):
    mesh = plsc.VectorSubcoreMesh(core_axis_name="c", subcore_axis_name="s",
                                  num_cores=2, num_subcores=16)
    f = pl.kernel(
        _body,
        out_type=jax.ShapeDtypeStruct((B * NUM_CROPS, C_IN, CH, CW),
                                      jnp.float32),
        mesh=mesh,
        compiler_params=pltpu.CompilerParams(use_tc_tiling_on_sc=True,
                                             needs_layout_passes=False),
        scratch_types=[
            pltpu.VMEM((B * NUM_CROPS * 2 + LANES,), jnp.int32),
            pltpu.VMEM((IR, W), jnp.float32),
            pltpu.VMEM((IR, W), jnp.float32),
            pltpu.VMEM((KR, CW), jnp.float32),
            pltpu.VMEM((KR, CW), jnp.float32),
            pltpu.SemaphoreType.DMA,
            pltpu.SemaphoreType.DMA,
            pltpu.SemaphoreType.DMA,
            pltpu.SemaphoreType.DMA,
        ],
    )
    return f(crop_inds.reshape(-1).astype(jnp.int32), inputs)


# speedup vs baseline: 2.4350x; 1.0964x over previous
import jax
import jax.numpy as jnp
from jax import lax
from jax.experimental import pallas as pl
from jax.experimental.pallas import tpu as pltpu
from jax.experimental.pallas import tpu_sc as plsc

B = 32
C_IN = 3
H = 512
W = 512
CH = 448
CW = 448
NUM_CROPS = 2

NW = 32
TRIPLES = B * NUM_CROPS * C_IN
PER_W = TRIPLES // NW
KR = 56
IR = KR + 8
NCHUNK = CH // KR
NU = PER_W * NCHUNK
LANES = 16
NJ = CW // LANES


def _body(inds_hbm, in_hbm, out_hbm, inds_v,
          ib0, ib1, ob0, ob1, isem0, isem1, osem0, osem1):
    wid = lax.axis_index("s") * 2 + lax.axis_index("c")
    pltpu.sync_copy(inds_hbm, inds_v.at[pl.ds(0, B * NUM_CROPS * 2)])
    lanes = lax.iota(jnp.int32, LANES)
    ibufs, obufs = (ib0, ib1), (ob0, ob1)
    isems, osems = (isem0, isem1), (osem0, osem1)

    def params(u):
        j = u // NCHUNK
        k = u % NCHUNK
        t = wid * PER_W + j
        b = t // (NUM_CROPS * C_IN)
        r = t % (NUM_CROPS * C_IN)
        n = r // C_IN
        c = r % C_IN
        hw = inds_v[pl.ds((b * NUM_CROPS + n) * 2, LANES)]
        h0 = hw[0]
        w0 = hw[1]
        h8 = pl.multiple_of((h0 // 8) * 8, 8)
        return b, n, c, k, h8, h0 - h8, w0

    def in_copy(u, s):
        b, _, c, k, h8, _, _ = params(u)
        return pltpu.make_async_copy(
            in_hbm.at[b, c, pl.ds(h8 + k * KR, IR), :],
            ibufs[s], isems[s])

    def out_copy(u, s):
        b, n, c, k, _, _, _ = params(u)
        return pltpu.make_async_copy(
            obufs[s],
            out_hbm.at[b * NUM_CROPS + n, c, pl.ds(k * KR, KR), :],
            osems[s])

    def compute(u, s):
        _, _, _, _, _, dh, w0 = params(u)
        ib, ob = ibufs[s], obufs[s]
        col0 = w0 + lanes

        @plsc.parallel_loop(0, KR, unroll=8)
        def _(rr):
            row = jnp.full((LANES,), rr + dh, jnp.int32)
            for jj in range(NJ):
                v = plsc.load_gather(ib, [row, col0 + jj * LANES])
                ob[rr, pl.ds(jj * LANES, LANES)] = v

    in_copy(0, 0).start()

    def pair_body(p, _):
        for s in range(2):
            u = 2 * p + s

            @pl.when(u + 1 < NU)
            def _():
                in_copy(u + 1, (s + 1) % 2).start()

            in_copy(u, s).wait()

            @pl.when(u >= 2)
            def _():
                out_copy(u - 2, s).wait()

            compute(u, s)
            out_copy(u, s).start()
        return 0

    lax.fori_loop(0, NU // 2, pair_body, 0)
    out_copy(NU - 2, 0).wait()
    out_copy(NU - 1, 1).wait()


def kernel(inputs, crop_inds):
    mesh = plsc.VectorSubcoreMesh(core_axis_name="c", subcore_axis_name="s",
                                  num_cores=2, num_subcores=16)
    f = pl.kernel(
        _body,
        out_type=jax.ShapeDtypeStruct((B * NUM_CROPS, C_IN, CH, CW),
                                      jnp.float32),
        mesh=mesh,
        compiler_params=pltpu.CompilerParams(use_tc_tiling_on_sc=True,
                                             needs_layout_passes=False),
        scratch_types=[
            pltpu.VMEM((B * NUM_CROPS * 2 + LANES,), jnp.int32),
            pltpu.VMEM((IR, W), jnp.float32),
            pltpu.VMEM((IR, W), jnp.float32),
            pltpu.VMEM((KR, CW), jnp.float32),
            pltpu.VMEM((KR, CW), jnp.float32),
            pltpu.SemaphoreType.DMA,
            pltpu.SemaphoreType.DMA,
            pltpu.SemaphoreType.DMA,
            pltpu.SemaphoreType.DMA,
        ],
    )
    return f(crop_inds.reshape(-1).astype(jnp.int32), inputs)
